# baseline (speedup 1.0000x reference)
import functools

import jax
import jax.numpy as jnp
from jax import lax
from jax.experimental import pallas as pl
from jax.experimental.pallas import tpu as pltpu
from jax.experimental.pallas import tpu_sc as plsc

N = 10000
B = 4096
EPP = 320000
EPQ = 90112
H = 128
PED = 64
CD = 15

NC = 2
NS = 16
NW = NC * NS
CW = 128

EPP_PAD = 327680
EW_PP = EPP_PAD // NW
CH_PP = EW_PP // CW
EPQ_PAD = 98304
EW_PQ = EPQ_PAD // NW
CH_PQ = EW_PQ // CW

ACC_P = 10112
ZP = ACC_P // NS
ACC_Q = 4224
ZQ = ACC_Q // NS
OQ = B // NS

IDS_PAD = 10240
IW = IDS_PAD // NW
DW = 16

PBLK = 2000
PGRID = N // PBLK


def _f32(shape):
    return jax.ShapeDtypeStruct(shape, jnp.float32)


@functools.lru_cache(maxsize=None)
def _sc_emb_deg():
    mesh = plsc.VectorSubcoreMesh(core_axis_name="c", subcore_axis_name="s")

    @functools.partial(
        pl.kernel, mesh=mesh,
        out_type=(_f32((IDS_PAD, H)), _f32((NC, ACC_P, H)), _f32((NC, B, H))),
        scratch_types=[
            pltpu.MemorySpace.VMEM_SHARED((ACC_P, H), jnp.float32),
            pltpu.MemorySpace.VMEM((IW,), jnp.int32),
            pltpu.MemorySpace.VMEM((CH_PP, CW), jnp.int32),
            pltpu.MemorySpace.VMEM((CH_PQ, CW), jnp.int32),
            pltpu.MemorySpace.VMEM((80, H), jnp.float32),
            pltpu.MemorySpace.VMEM((CW, H), jnp.float32),
            pltpu.SemaphoreType.DMA,
        ],
    )
    def k(emb_hbm, ids_hbm, dpp_hbm, dpq_hbm, ones_hbm, zeros_hbm,
          emb_out, dp_out, dq_out,
          acc, idsv, dppv, dpqv, ebuf, onesv, sem):
        cc = lax.axis_index("c")
        ss = lax.axis_index("s")
        wid = ss * NC + cc
        pltpu.sync_copy(zeros_hbm, acc.at[pl.ds(ss * ZP, ZP)])
        pltpu.sync_copy(ids_hbm.at[pl.ds(wid * IW, IW)], idsv)
        pltpu.sync_copy(dpp_hbm.at[pl.ds(wid * CH_PP, CH_PP)], dppv)
        pltpu.sync_copy(dpq_hbm.at[pl.ds(wid * CH_PQ, CH_PQ)], dpqv)
        pltpu.sync_copy(ones_hbm, onesv)
        for c in range(IW // 80):
            pltpu.async_copy(emb_hbm.at[idsv.at[pl.ds(c * 80, 80)]], ebuf,
                             sem).wait()
            pltpu.sync_copy(ebuf, emb_out.at[pl.ds(wid * IW + c * 80, 80)])
        plsc.subcore_barrier()

        def deg_scatter(dstv_ref, nch):
            def group(i, _):
                for j in range(8):
                    pltpu.async_copy(onesv, acc.at[dstv_ref.at[i * 8 + j]],
                                     sem, add=True)
                for j in range(8):
                    pltpu.make_async_copy(onesv, acc.at[dstv_ref.at[i * 8 + j]],
                                          sem).wait()
                return ()

            lax.fori_loop(0, nch // 8, group, ())

        deg_scatter(dppv, CH_PP)
        plsc.subcore_barrier()
        pltpu.sync_copy(acc.at[pl.ds(ss * ZP, ZP)],
                        dp_out.at[cc, pl.ds(ss * ZP, ZP)])
        plsc.subcore_barrier()
        pltpu.sync_copy(zeros_hbm.at[pl.ds(0, ZQ)], acc.at[pl.ds(ss * ZQ, ZQ)])
        plsc.subcore_barrier()
        deg_scatter(dpqv, CH_PQ)
        plsc.subcore_barrier()
        pltpu.sync_copy(acc.at[pl.ds(ss * OQ, OQ)],
                        dq_out.at[cc, pl.ds(ss * OQ, OQ)])

    return k


HCH = CH_PP // 2


@functools.lru_cache(maxsize=None)
def _sc_seg():
    mesh = plsc.VectorSubcoreMesh(core_axis_name="c", subcore_axis_name="s")

    @functools.partial(
        pl.kernel, mesh=mesh,
        out_type=(_f32((NC, ACC_P, H)), _f32((NC, B, H))),
        scratch_types=[
            pltpu.MemorySpace.VMEM_SHARED((ACC_P, H), jnp.float32),
            pltpu.MemorySpace.VMEM((HCH * CW,), jnp.int32),
            pltpu.MemorySpace.VMEM((HCH, CW), jnp.int32),
            pltpu.MemorySpace.VMEM((CW, H), jnp.float32),
            pltpu.MemorySpace.VMEM((CW, H), jnp.float32),
            pltpu.SemaphoreType.DMA,
            pltpu.SemaphoreType.DMA,
            pltpu.SemaphoreType.DMA,
            pltpu.SemaphoreType.DMA,
        ],
    )
    def k(yp_hbm, yq_hbm, spp_hbm, dpp_hbm, spq_hbm, dpq_hbm, zeros_hbm,
          pp_out, pq_out,
          acc, srcv, dstv, r0, r1, s0, s1, t0, t1):
        cc = lax.axis_index("c")
        ss = lax.axis_index("s")
        wid = ss * NC + cc
        rows = (r0, r1)
        sems = (s0, s1)
        ssems = (t0, t1)

        def load_idx(src_hbm, src_off, dst_hbm, dst_off, nch):
            pltpu.sync_copy(src_hbm.at[pl.ds(src_off, nch * CW)],
                            srcv.at[pl.ds(0, nch * CW)])
            pltpu.sync_copy(dst_hbm.at[pl.ds(dst_off, nch)],
                            dstv.at[pl.ds(0, nch)])

        def run_edges(y_hbm, nch):
            def body(i, _):
                for b in range(2):
                    c = i * 2 + b

                    @pl.when(c >= 2)
                    def _():
                        pltpu.make_async_copy(
                            rows[b], acc.at[dstv.at[c - 2]], ssems[b]).wait()

                    pltpu.async_copy(y_hbm.at[srcv.at[pl.ds(c * CW, CW)]],
                                     rows[b], sems[b]).wait()
                    pltpu.async_copy(rows[b], acc.at[dstv.at[c]], ssems[b],
                                     add=True)
                return ()

            lax.fori_loop(0, nch // 2, body, ())
            for b in range(2):
                pltpu.make_async_copy(rows[b], acc.at[dstv.at[nch - 2 + b]],
                                      ssems[b]).wait()

        pltpu.sync_copy(zeros_hbm, acc.at[pl.ds(ss * ZP, ZP)])
        load_idx(spp_hbm, wid * EW_PP, dpp_hbm, wid * CH_PP, HCH)
        plsc.subcore_barrier()
        run_edges(yp_hbm, HCH)
        load_idx(spp_hbm, wid * EW_PP + HCH * CW, dpp_hbm,
                 wid * CH_PP + HCH, HCH)
        run_edges(yp_hbm, HCH)
        plsc.subcore_barrier()
        pltpu.sync_copy(acc.at[pl.ds(ss * ZP, ZP)],
                        pp_out.at[cc, pl.ds(ss * ZP, ZP)])
        plsc.subcore_barrier()
        pltpu.sync_copy(zeros_hbm.at[pl.ds(0, ZQ)], acc.at[pl.ds(ss * ZQ, ZQ)])
        load_idx(spq_hbm, wid * EW_PQ, dpq_hbm, wid * CH_PQ, CH_PQ)
        plsc.subcore_barrier()
        run_edges(yq_hbm, CH_PQ)
        plsc.subcore_barrier()
        pltpu.sync_copy(acc.at[pl.ds(ss * OQ, OQ)],
                        pq_out.at[cc, pl.ds(ss * OQ, OQ)])

    return k


def _dot(a, b):
    return jnp.dot(a, b, preferred_element_type=jnp.float32)


def _relu(x):
    return jnp.maximum(x, 0.0)


def _tc_p_enc(emb_rows, w_enc, b_enc, wpp0, wpq0):
    def body(e_ref, w_ref, b_ref, wpp_ref, wpq_ref, xp_ref, yp_ref, ypq_ref):
        xp = _relu(_dot(e_ref[...], w_ref[...]) + b_ref[...])
        xp_ref[...] = xp
        yp_ref[...] = _dot(xp, wpp_ref[...])
        ypq_ref[...] = _dot(xp, wpq_ref[...])

    return pl.pallas_call(
        body,
        grid=(PGRID,),
        in_specs=[
            pl.BlockSpec((PBLK, H), lambda i: (i, 0)),
            pl.BlockSpec((H, H), lambda i: (0, 0)),
            pl.BlockSpec((1, H), lambda i: (0, 0)),
            pl.BlockSpec((H, H), lambda i: (0, 0)),
            pl.BlockSpec((H, H), lambda i: (0, 0)),
        ],
        out_specs=[pl.BlockSpec((PBLK, H), lambda i: (i, 0))] * 3,
        out_shape=[_f32((N, H))] * 3,
    )(emb_rows, w_enc, b_enc, wpp0, wpq0)


def _tc_q_enc(qf, w_enc, b_enc, dp_parts, dq_parts):
    def body(q_ref, w_ref, b_ref, dp_ref, dq_ref, xq_ref, ip_ref, iq_ref):
        xq_ref[...] = _relu(_dot(q_ref[...], w_ref[...]) + b_ref[...])
        ip_ref[...] = 1.0 / jnp.maximum(dp_ref[0][:, :DW] + dp_ref[1][:, :DW],
                                        1.0)
        iq_ref[...] = 1.0 / jnp.maximum(dq_ref[0][:, :DW] + dq_ref[1][:, :DW],
                                        1.0)

    return pl.pallas_call(
        body,
        out_shape=[_f32((B, H)), _f32((ACC_P, DW)), _f32((B, DW))],
    )(qf, w_enc, b_enc, dp_parts, dq_parts)


def _tc_p_upd(xp, pp_parts, invp, w_self, b, wpp_n, wpq_n):
    def body(x_ref, p_ref, i_ref, w_ref, b_ref, wpp_ref, wpq_ref,
             xo_ref, yp_ref, ypq_ref):
        x = x_ref[...]
        agg = (p_ref[0] + p_ref[1]) * i_ref[:, 0:1]
        x2 = x + _relu(_dot(x, w_ref[...]) + agg + b_ref[...])
        xo_ref[...] = x2
        yp_ref[...] = _dot(x2, wpp_ref[...])
        ypq_ref[...] = _dot(x2, wpq_ref[...])

    return pl.pallas_call(
        body,
        grid=(PGRID,),
        in_specs=[
            pl.BlockSpec((PBLK, H), lambda i: (i, 0)),
            pl.BlockSpec((NC, PBLK, H), lambda i: (0, i, 0)),
            pl.BlockSpec((PBLK, DW), lambda i: (i, 0)),
            pl.BlockSpec((H, H), lambda i: (0, 0)),
            pl.BlockSpec((1, H), lambda i: (0, 0)),
            pl.BlockSpec((H, H), lambda i: (0, 0)),
            pl.BlockSpec((H, H), lambda i: (0, 0)),
        ],
        out_specs=[pl.BlockSpec((PBLK, H), lambda i: (i, 0))] * 3,
        out_shape=[_f32((N, H))] * 3,
    )(xp, pp_parts, invp, w_self, b, wpp_n, wpq_n)


def _tc_q_upd(xq, pq_parts, invq, w_self, b):
    def body(x_ref, p_ref, i_ref, w_ref, b_ref, xo_ref):
        x = x_ref[...]
        agg = (p_ref[0] + p_ref[1]) * i_ref[:, 0:1]
        xo_ref[...] = x + _relu(_dot(x, w_ref[...]) + agg + b_ref[...])

    return pl.pallas_call(body, out_shape=_f32((B, H)))(
        xq, pq_parts, invq, w_self, b)


def _tc_final(xq, pq_parts, invq, w_self, b, wb1, bb1, wb2, bb2,
              ww1, bw1, ww2, bw2):
    def body(x_ref, p_ref, i_ref, w_ref, b_ref, wb1_ref, bb1_ref, wb2_ref,
             bb2_ref, ww1_ref, bw1_ref, ww2_ref, bw2_ref, o_ref):
        x = x_ref[...]
        agg = (p_ref[0] + p_ref[1]) * i_ref[:, 0:1]
        x2 = x + _relu(_dot(x, w_ref[...]) + agg + b_ref[...])
        hb = _relu(_dot(x2, wb1_ref[...]) + bb1_ref[...])
        hw = _relu(_dot(x2, ww1_ref[...]) + bw1_ref[...])
        ob = _dot(hb, wb2_ref[...]) + bb2_ref[...]
        ow = _dot(hw, ww2_ref[...]) + bw2_ref[...]
        o_ref[...] = jnp.concatenate([ob, ow], axis=1)

    return pl.pallas_call(body, out_shape=_f32((B, 2 * 8)))(
        xq, pq_parts, invq, w_self, b, wb1, bb1, wb2, bb2, ww1, bw1, ww2, bw2)


def kernel(player_ids, query_feat, edge_index_pp, src_pq, dst_pq, emb_table,
           W_p_enc, b_p_enc, W_q_enc, b_q_enc, W_pp, W_pq, W_self_p, W_self_q,
           b_p, b_q, Wb1, bb1, Wb2, bb2, Ww1, bw1, Ww2, bw2):
    i32 = jnp.int32
    ids_pad = jnp.concatenate(
        [player_ids.astype(i32), (jnp.arange(IDS_PAD - N, dtype=i32) * 37) % N])
    npad = EPP_PAD - EPP
    src_pp_pad = jnp.concatenate(
        [edge_index_pp[0].astype(i32),
         (jnp.arange(npad, dtype=i32) * 37) % N])
    dst_pp_pad = jnp.concatenate(
        [edge_index_pp[1].astype(i32),
         N + (jnp.arange(npad, dtype=i32) % 32)])
    dpp2d = dst_pp_pad.reshape(NW * CH_PP, CW)
    qpad = EPQ_PAD - EPQ
    spq = jnp.concatenate(
        [src_pq.astype(i32), (jnp.arange(qpad, dtype=i32) * 37) % N])
    dst_pq_pad = jnp.concatenate(
        [dst_pq.astype(i32), B + (jnp.arange(qpad, dtype=i32) % 32)])
    dpq2d = dst_pq_pad.reshape(NW * CH_PQ, CW)

    zeros_h = jnp.zeros((ZP, H), jnp.float32)
    ones_h = jnp.ones((CW, H), jnp.float32)

    emb_pad = jnp.pad(emb_table, ((0, 0), (0, H - PED)))
    wp_pad = jnp.pad(W_p_enc, ((0, H - PED), (0, 0)))
    emb_rows, dp_parts, dq_parts = _sc_emb_deg()(
        emb_pad, ids_pad, dpp2d, dpq2d, ones_h, zeros_h)

    xp, yp, ypq = _tc_p_enc(emb_rows, wp_pad,
                            b_p_enc.reshape(1, H), W_pp[0], W_pq[0])
    qf_pad = jnp.pad(query_feat, ((0, 0), (0, 1)))
    wq_pad = jnp.pad(W_q_enc, ((0, 1), (0, 0)))
    xq, invp, invq = _tc_q_enc(qf_pad, wq_pad, b_q_enc.reshape(1, H),
                               dp_parts, dq_parts)

    wb2p = jnp.pad(Wb2, ((0, 0), (0, 7)))
    ww2p = jnp.pad(Ww2, ((0, 0), (0, 7)))
    bb2p = jnp.pad(bb2.reshape(1, 1), ((0, 0), (0, 7)))
    bw2p = jnp.pad(bw2.reshape(1, 1), ((0, 0), (0, 7)))

    out = None
    for l in range(3):
        pp_parts, pq_parts = _sc_seg()(
            yp, ypq, src_pp_pad, dpp2d, spq, dpq2d, zeros_h)
        if l < 2:
            xp, yp, ypq = _tc_p_upd(xp, pp_parts, invp, W_self_p[l],
                                    b_p[l].reshape(1, H),
                                    W_pp[l + 1], W_pq[l + 1])
            xq = _tc_q_upd(xq, pq_parts, invq, W_self_q[l],
                           b_q[l].reshape(1, H))
        else:
            out = _tc_final(xq, pq_parts, invq, W_self_q[2],
                            b_q[2].reshape(1, H),
                            Wb1, bb1.reshape(1, H // 2), wb2p, bb2p,
                            Ww1, bw1.reshape(1, H // 2), ww2p, bw2p)

    return (out[:, 0:1], out[:, 8:9])


# speedup vs baseline: 1.1256x; 1.1256x over previous
import functools

import jax
import jax.numpy as jnp
from jax import lax
from jax.experimental import pallas as pl
from jax.experimental.pallas import tpu as pltpu
from jax.experimental.pallas import tpu_sc as plsc

N = 10000
B = 4096
EPP = 320000
EPQ = 90112
H = 128
PED = 64
CD = 15

NC = 2
NS = 16
NW = NC * NS
CW = 128

EPP_PAD = 327680
EW_PP = EPP_PAD // NW
CH_PP = EW_PP // CW
EPQ_PAD = 98304
EW_PQ = EPQ_PAD // NW
CH_PQ = EW_PQ // CW

ACC_P = 10112
ZP = ACC_P // NS
ACC_Q = 4224
ZQ = ACC_Q // NS
OQ = B // NS

IDS_PAD = 10240
IW = IDS_PAD // NW
DW = 16

PBLK = 2000
PGRID = N // PBLK


def _f32(shape):
    return jax.ShapeDtypeStruct(shape, jnp.float32)


@functools.lru_cache(maxsize=None)
def _sc_emb_deg():
    mesh = plsc.VectorSubcoreMesh(core_axis_name="c", subcore_axis_name="s")

    @functools.partial(
        pl.kernel, mesh=mesh,
        out_type=(_f32((IDS_PAD, H)), _f32((NC, ACC_P, H)), _f32((NC, B, H))),
        scratch_types=[
            pltpu.MemorySpace.VMEM_SHARED((ACC_P, H), jnp.float32),
            pltpu.MemorySpace.VMEM((IW,), jnp.int32),
            pltpu.MemorySpace.VMEM((CH_PP, CW), jnp.int32),
            pltpu.MemorySpace.VMEM((CH_PQ, CW), jnp.int32),
            pltpu.MemorySpace.VMEM((80, H), jnp.float32),
            pltpu.MemorySpace.VMEM((CW, H), jnp.float32),
            pltpu.SemaphoreType.DMA,
        ],
    )
    def k(emb_hbm, ids_hbm, dpp_hbm, dpq_hbm, ones_hbm, zeros_hbm,
          emb_out, dp_out, dq_out,
          acc, idsv, dppv, dpqv, ebuf, onesv, sem):
        cc = lax.axis_index("c")
        ss = lax.axis_index("s")
        wid = ss * NC + cc
        pltpu.sync_copy(zeros_hbm, acc.at[pl.ds(ss * ZP, ZP)])
        pltpu.sync_copy(ids_hbm.at[pl.ds(wid * IW, IW)], idsv)
        pltpu.sync_copy(dpp_hbm.at[pl.ds(wid * CH_PP, CH_PP)], dppv)
        pltpu.sync_copy(dpq_hbm.at[pl.ds(wid * CH_PQ, CH_PQ)], dpqv)
        pltpu.sync_copy(ones_hbm, onesv)
        for c in range(IW // 80):
            pltpu.async_copy(emb_hbm.at[idsv.at[pl.ds(c * 80, 80)]], ebuf,
                             sem).wait()
            pltpu.sync_copy(ebuf, emb_out.at[pl.ds(wid * IW + c * 80, 80)])
        plsc.subcore_barrier()

        def deg_scatter(dstv_ref, nch):
            def group(i, _):
                for j in range(8):
                    pltpu.async_copy(onesv, acc.at[dstv_ref.at[i * 8 + j]],
                                     sem, add=True)
                for j in range(8):
                    pltpu.make_async_copy(onesv, acc.at[dstv_ref.at[i * 8 + j]],
                                          sem).wait()
                return ()

            lax.fori_loop(0, nch // 8, group, ())

        deg_scatter(dppv, CH_PP)
        plsc.subcore_barrier()
        pltpu.sync_copy(acc.at[pl.ds(ss * ZP, ZP)],
                        dp_out.at[cc, pl.ds(ss * ZP, ZP)])
        plsc.subcore_barrier()
        pltpu.sync_copy(zeros_hbm.at[pl.ds(0, ZQ)], acc.at[pl.ds(ss * ZQ, ZQ)])
        plsc.subcore_barrier()
        deg_scatter(dpqv, CH_PQ)
        plsc.subcore_barrier()
        pltpu.sync_copy(acc.at[pl.ds(ss * OQ, OQ)],
                        dq_out.at[cc, pl.ds(ss * OQ, OQ)])

    return k


HCH = CH_PP // 2


@functools.lru_cache(maxsize=None)
def _sc_seg():
    mesh = plsc.VectorSubcoreMesh(core_axis_name="c", subcore_axis_name="s")

    @functools.partial(
        pl.kernel, mesh=mesh,
        out_type=(_f32((NC, ACC_P, H)), _f32((NC, B, H))),
        scratch_types=[
            pltpu.MemorySpace.VMEM_SHARED((ACC_P, H), jnp.float32),
            pltpu.MemorySpace.VMEM((HCH * CW,), jnp.int32),
            pltpu.MemorySpace.VMEM((HCH, CW), jnp.int32),
            pltpu.MemorySpace.VMEM((CW, H), jnp.float32),
            pltpu.MemorySpace.VMEM((CW, H), jnp.float32),
            pltpu.SemaphoreType.DMA,
            pltpu.SemaphoreType.DMA,
            pltpu.SemaphoreType.DMA,
            pltpu.SemaphoreType.DMA,
        ],
    )
    def k(yp_hbm, yq_hbm, spp_hbm, dpp_hbm, spq_hbm, dpq_hbm, zeros_hbm,
          pp_out, pq_out,
          acc, srcv, dstv, r0, r1, s0, s1, t0, t1):
        cc = lax.axis_index("c")
        ss = lax.axis_index("s")
        wid = ss * NC + cc
        rows = (r0, r1)
        sems = (s0, s1)
        ssems = (t0, t1)

        def load_idx(src_hbm, src_off, dst_hbm, dst_off, nch):
            pltpu.sync_copy(src_hbm.at[pl.ds(src_off, nch * CW)],
                            srcv.at[pl.ds(0, nch * CW)])
            pltpu.sync_copy(dst_hbm.at[pl.ds(dst_off, nch)],
                            dstv.at[pl.ds(0, nch)])

        def run_edges(y_hbm, nch):
            def issue(c, b):
                pltpu.async_copy(y_hbm.at[srcv.at[pl.ds(c * CW, CW)]],
                                 rows[b], sems[b])

            issue(0, 0)
            issue(1, 1)

            def body(i, _):
                for b in range(2):
                    c = i * 2 + b
                    pltpu.make_async_copy(
                        y_hbm.at[srcv.at[pl.ds(c * CW, CW)]],
                        rows[b], sems[b]).wait()
                    pltpu.sync_copy(rows[b], acc.at[dstv.at[c]], add=True)

                    @pl.when(c + 2 < nch)
                    def _():
                        issue(c + 2, b)
                return ()

            lax.fori_loop(0, nch // 2, body, ())

        pltpu.sync_copy(zeros_hbm, acc.at[pl.ds(ss * ZP, ZP)])
        load_idx(spp_hbm, wid * EW_PP, dpp_hbm, wid * CH_PP, HCH)
        plsc.subcore_barrier()
        run_edges(yp_hbm, HCH)
        load_idx(spp_hbm, wid * EW_PP + HCH * CW, dpp_hbm,
                 wid * CH_PP + HCH, HCH)
        run_edges(yp_hbm, HCH)
        plsc.subcore_barrier()
        pltpu.sync_copy(acc.at[pl.ds(ss * ZP, ZP)],
                        pp_out.at[cc, pl.ds(ss * ZP, ZP)])
        plsc.subcore_barrier()
        pltpu.sync_copy(zeros_hbm.at[pl.ds(0, ZQ)], acc.at[pl.ds(ss * ZQ, ZQ)])
        load_idx(spq_hbm, wid * EW_PQ, dpq_hbm, wid * CH_PQ, CH_PQ)
        plsc.subcore_barrier()
        run_edges(yq_hbm, CH_PQ)
        plsc.subcore_barrier()
        pltpu.sync_copy(acc.at[pl.ds(ss * OQ, OQ)],
                        pq_out.at[cc, pl.ds(ss * OQ, OQ)])

    return k


def _dot(a, b):
    return jnp.dot(a, b, preferred_element_type=jnp.float32)


def _relu(x):
    return jnp.maximum(x, 0.0)


def _tc_p_enc(emb_rows, w_enc, b_enc, wpp0, wpq0):
    def body(e_ref, w_ref, b_ref, wpp_ref, wpq_ref, xp_ref, yp_ref, ypq_ref):
        xp = _relu(_dot(e_ref[...], w_ref[...]) + b_ref[...])
        xp_ref[...] = xp
        yp_ref[...] = _dot(xp, wpp_ref[...])
        ypq_ref[...] = _dot(xp, wpq_ref[...])

    return pl.pallas_call(
        body,
        grid=(PGRID,),
        in_specs=[
            pl.BlockSpec((PBLK, H), lambda i: (i, 0)),
            pl.BlockSpec((H, H), lambda i: (0, 0)),
            pl.BlockSpec((1, H), lambda i: (0, 0)),
            pl.BlockSpec((H, H), lambda i: (0, 0)),
            pl.BlockSpec((H, H), lambda i: (0, 0)),
        ],
        out_specs=[pl.BlockSpec((PBLK, H), lambda i: (i, 0))] * 3,
        out_shape=[_f32((N, H))] * 3,
    )(emb_rows, w_enc, b_enc, wpp0, wpq0)


def _tc_q_enc(qf, w_enc, b_enc, dp_parts, dq_parts):
    def body(q_ref, w_ref, b_ref, dp_ref, dq_ref, xq_ref, ip_ref, iq_ref):
        xq_ref[...] = _relu(_dot(q_ref[...], w_ref[...]) + b_ref[...])
        ip_ref[...] = 1.0 / jnp.maximum(dp_ref[0][:, :DW] + dp_ref[1][:, :DW],
                                        1.0)
        iq_ref[...] = 1.0 / jnp.maximum(dq_ref[0][:, :DW] + dq_ref[1][:, :DW],
                                        1.0)

    return pl.pallas_call(
        body,
        out_shape=[_f32((B, H)), _f32((ACC_P, DW)), _f32((B, DW))],
    )(qf, w_enc, b_enc, dp_parts, dq_parts)


def _tc_p_upd(xp, pp_parts, invp, w_self, b, wpp_n, wpq_n):
    def body(x_ref, p_ref, i_ref, w_ref, b_ref, wpp_ref, wpq_ref,
             xo_ref, yp_ref, ypq_ref):
        x = x_ref[...]
        agg = (p_ref[0] + p_ref[1]) * i_ref[:, 0:1]
        x2 = x + _relu(_dot(x, w_ref[...]) + agg + b_ref[...])
        xo_ref[...] = x2
        yp_ref[...] = _dot(x2, wpp_ref[...])
        ypq_ref[...] = _dot(x2, wpq_ref[...])

    return pl.pallas_call(
        body,
        grid=(PGRID,),
        in_specs=[
            pl.BlockSpec((PBLK, H), lambda i: (i, 0)),
            pl.BlockSpec((NC, PBLK, H), lambda i: (0, i, 0)),
            pl.BlockSpec((PBLK, DW), lambda i: (i, 0)),
            pl.BlockSpec((H, H), lambda i: (0, 0)),
            pl.BlockSpec((1, H), lambda i: (0, 0)),
            pl.BlockSpec((H, H), lambda i: (0, 0)),
            pl.BlockSpec((H, H), lambda i: (0, 0)),
        ],
        out_specs=[pl.BlockSpec((PBLK, H), lambda i: (i, 0))] * 3,
        out_shape=[_f32((N, H))] * 3,
    )(xp, pp_parts, invp, w_self, b, wpp_n, wpq_n)


def _tc_q_upd(xq, pq_parts, invq, w_self, b):
    def body(x_ref, p_ref, i_ref, w_ref, b_ref, xo_ref):
        x = x_ref[...]
        agg = (p_ref[0] + p_ref[1]) * i_ref[:, 0:1]
        xo_ref[...] = x + _relu(_dot(x, w_ref[...]) + agg + b_ref[...])

    return pl.pallas_call(body, out_shape=_f32((B, H)))(
        xq, pq_parts, invq, w_self, b)


def _tc_final(xq, pq_parts, invq, w_self, b, wb1, bb1, wb2, bb2,
              ww1, bw1, ww2, bw2):
    def body(x_ref, p_ref, i_ref, w_ref, b_ref, wb1_ref, bb1_ref, wb2_ref,
             bb2_ref, ww1_ref, bw1_ref, ww2_ref, bw2_ref, o_ref):
        x = x_ref[...]
        agg = (p_ref[0] + p_ref[1]) * i_ref[:, 0:1]
        x2 = x + _relu(_dot(x, w_ref[...]) + agg + b_ref[...])
        hb = _relu(_dot(x2, wb1_ref[...]) + bb1_ref[...])
        hw = _relu(_dot(x2, ww1_ref[...]) + bw1_ref[...])
        ob = _dot(hb, wb2_ref[...]) + bb2_ref[...]
        ow = _dot(hw, ww2_ref[...]) + bw2_ref[...]
        o_ref[...] = jnp.concatenate([ob, ow], axis=1)

    return pl.pallas_call(body, out_shape=_f32((B, 2 * 8)))(
        xq, pq_parts, invq, w_self, b, wb1, bb1, wb2, bb2, ww1, bw1, ww2, bw2)


def kernel(player_ids, query_feat, edge_index_pp, src_pq, dst_pq, emb_table,
           W_p_enc, b_p_enc, W_q_enc, b_q_enc, W_pp, W_pq, W_self_p, W_self_q,
           b_p, b_q, Wb1, bb1, Wb2, bb2, Ww1, bw1, Ww2, bw2):
    i32 = jnp.int32
    ids_pad = jnp.concatenate(
        [player_ids.astype(i32), (jnp.arange(IDS_PAD - N, dtype=i32) * 37) % N])
    npad = EPP_PAD - EPP
    src_pp_pad = jnp.concatenate(
        [edge_index_pp[0].astype(i32),
         (jnp.arange(npad, dtype=i32) * 37) % N])
    dst_pp_pad = jnp.concatenate(
        [edge_index_pp[1].astype(i32),
         N + (jnp.arange(npad, dtype=i32) % 32)])
    dpp2d = dst_pp_pad.reshape(NW * CH_PP, CW)
    qpad = EPQ_PAD - EPQ
    spq = jnp.concatenate(
        [src_pq.astype(i32), (jnp.arange(qpad, dtype=i32) * 37) % N])
    dst_pq_pad = jnp.concatenate(
        [dst_pq.astype(i32), B + (jnp.arange(qpad, dtype=i32) % 32)])
    dpq2d = dst_pq_pad.reshape(NW * CH_PQ, CW)

    zeros_h = jnp.zeros((ZP, H), jnp.float32)
    ones_h = jnp.ones((CW, H), jnp.float32)

    emb_pad = jnp.pad(emb_table, ((0, 0), (0, H - PED)))
    wp_pad = jnp.pad(W_p_enc, ((0, H - PED), (0, 0)))
    emb_rows, dp_parts, dq_parts = _sc_emb_deg()(
        emb_pad, ids_pad, dpp2d, dpq2d, ones_h, zeros_h)

    xp, yp, ypq = _tc_p_enc(emb_rows, wp_pad,
                            b_p_enc.reshape(1, H), W_pp[0], W_pq[0])
    qf_pad = jnp.pad(query_feat, ((0, 0), (0, 1)))
    wq_pad = jnp.pad(W_q_enc, ((0, 1), (0, 0)))
    xq, invp, invq = _tc_q_enc(qf_pad, wq_pad, b_q_enc.reshape(1, H),
                               dp_parts, dq_parts)

    wb2p = jnp.pad(Wb2, ((0, 0), (0, 7)))
    ww2p = jnp.pad(Ww2, ((0, 0), (0, 7)))
    bb2p = jnp.pad(bb2.reshape(1, 1), ((0, 0), (0, 7)))
    bw2p = jnp.pad(bw2.reshape(1, 1), ((0, 0), (0, 7)))

    out = None
    for l in range(3):
        pp_parts, pq_parts = _sc_seg()(
            yp, ypq, src_pp_pad, dpp2d, spq, dpq2d, zeros_h)
        if l < 2:
            xp, yp, ypq = _tc_p_upd(xp, pp_parts, invp, W_self_p[l],
                                    b_p[l].reshape(1, H),
                                    W_pp[l + 1], W_pq[l + 1])
            xq = _tc_q_upd(xq, pq_parts, invq, W_self_q[l],
                           b_q[l].reshape(1, H))
        else:
            out = _tc_final(xq, pq_parts, invq, W_self_q[2],
                            b_q[2].reshape(1, H),
                            Wb1, bb1.reshape(1, H // 2), wb2p, bb2p,
                            Ww1, bw1.reshape(1, H // 2), ww2p, bw2p)

    return (out[:, 0:1], out[:, 8:9])
